# Initial kernel scaffold; baseline (speedup 1.0000x reference)
import jax
import jax.numpy as jnp
from jax.experimental import pallas as pl


def kernel(loc_data, conf_data, regr_data, priors, t_coords, t_labels, t_regr):
    raise NotImplementedError("write your pallas kernel here")


# speedup vs baseline: 10.4788x; 10.4788x over previous
import jax
import jax.numpy as jnp
from jax import lax
from jax.experimental import pallas as pl
from jax.experimental.pallas import tpu as pltpu

_THRESH = 0.5
_NEGPOS = 3.0
_VAR = 0.1
_CHUNK = 4096
_TILE = 4096


def _match_kernel(tc_ref, lbl_ref, trg_ref, pt_ref, conf_ref, g_ref, rt_ref,
                  btm_scr, bti_scr):
    O = tc_ref.shape[1]
    P = pt_ref.shape[1]
    nch = P // _CHUNK
    tc = tc_ref[0]
    ax1, ay1 = tc[:, 0:1], tc[:, 1:2]
    ax2, ay2 = tc[:, 2:3], tc[:, 3:4]
    area_a = (ax2 - ax1) * (ay2 - ay1)
    lblp1 = lbl_ref[0].astype(jnp.float32) + 1.0
    trg = trg_ref[0]
    tcx = (ax1 + ax2) * 0.5
    tcy = (ay1 + ay2) * 0.5
    row_i = lax.broadcasted_iota(jnp.int32, (O, _CHUNK), 0)
    lane_i = lax.broadcasted_iota(jnp.int32, (O, _CHUNK), 1)

    bp_max = jnp.full((O, 1), -1.0, jnp.float32)
    bp_idx = jnp.zeros((O, 1), jnp.int32)
    for c in range(nch):
        sl = slice(c * _CHUNK, (c + 1) * _CHUNK)
        pcx, pcy = pt_ref[0:1, sl], pt_ref[1:2, sl]
        pw, ph = pt_ref[2:3, sl], pt_ref[3:4, sl]
        bx1, by1 = pcx - pw / 2, pcy - ph / 2
        bx2, by2 = pcx + pw / 2, pcy + ph / 2
        wid = jnp.maximum(jnp.minimum(ax2, bx2) - jnp.maximum(ax1, bx1), 0.0)
        hei = jnp.maximum(jnp.minimum(ay2, by2) - jnp.maximum(ay1, by1), 0.0)
        inter = wid * hei
        area_b = (bx2 - bx1) * (by2 - by1)
        ovl = inter / (area_a + area_b - inter)
        cmax = jnp.max(ovl, axis=1, keepdims=True)
        cidx = jnp.min(jnp.where(ovl == cmax, lane_i, P), axis=1,
                       keepdims=True) + c * _CHUNK
        upd = cmax > bp_max
        bp_idx = jnp.where(upd, cidx, bp_idx)
        bp_max = jnp.where(upd, cmax, bp_max)
        btm = jnp.max(ovl, axis=0, keepdims=True)
        bti = jnp.min(jnp.where(ovl == btm, row_i, O), axis=0, keepdims=True)
        btm_scr[0:1, sl] = btm
        bti_scr[0:1, sl] = bti

    for c in range(nch):
        sl = slice(c * _CHUNK, (c + 1) * _CHUNK)
        btm = btm_scr[0:1, sl]
        bti = bti_scr[0:1, sl]
        lane_g = lane_i + c * _CHUNK
        F = lane_g == bp_idx
        ch_o = jnp.max(jnp.where(F, row_i, -1), axis=0, keepdims=True)
        forced = ch_o >= 0
        fidx = jnp.where(forced, ch_o, bti)
        fovl = jnp.where(forced, 2.0, btm)
        eq2 = row_i == fidx
        mlbl = jnp.sum(jnp.where(eq2, lblp1, 0.0), axis=0, keepdims=True)
        mcx = jnp.sum(jnp.where(eq2, tcx, 0.0), axis=0, keepdims=True)
        mcy = jnp.sum(jnp.where(eq2, tcy, 0.0), axis=0, keepdims=True)
        mrg = jnp.sum(jnp.where(eq2, trg, 0.0), axis=0, keepdims=True)
        conf = jnp.where(fovl < _THRESH, 0.0, mlbl)
        pcx, pcy = pt_ref[0:1, sl], pt_ref[1:2, sl]
        pw, ph = pt_ref[2:3, sl], pt_ref[3:4, sl]
        conf_ref[0, 0:1, sl] = conf.astype(jnp.int32)
        g_ref[0, 0:1, sl] = (mcx - pcx) / (_VAR * pw)
        g_ref[0, 1:2, sl] = (mcy - pcy) / (_VAR * ph)
        rt_ref[0, 0:1, sl] = mrg


def _stream_kernel(conf_ref, loc_ref, regr_ref, ct_ref, g_ref, rt_ref,
                   ce_ref, acc_ref):
    b = pl.program_id(0)
    t = pl.program_id(1)
    T, C = conf_ref.shape[1], conf_ref.shape[2]

    @pl.when(jnp.logical_and(b == 0, t == 0))
    def _init():
        acc_ref[...] = jnp.zeros_like(acc_ref)

    x = conf_ref[0]
    m = jnp.max(x, axis=1, keepdims=True)
    s = jnp.sum(jnp.exp(x - m), axis=1, keepdims=True)
    lse = m + jnp.log(s)
    ct = ct_ref[0]
    cls_i = lax.broadcasted_iota(jnp.int32, (T, C), 1)
    gathered = jnp.sum(jnp.where(cls_i == ct, x, 0.0), axis=1, keepdims=True)
    ce = lse - gathered
    pos = ct > 0
    posf = pos.astype(jnp.float32)
    ce_ref[0] = jnp.where(pos, 0.0, ce)

    d = loc_ref[0] - g_ref[0]
    ad = jnp.abs(d)
    sl1 = jnp.where(ad < 1.0, 0.5 * d * d, ad - 0.5)
    part_l = jnp.sum(sl1 * posf)
    part_r = jnp.sum(jnp.abs(regr_ref[0] - rt_ref[0]) * posf)
    part_ce = jnp.sum(ce * posf)
    npos = jnp.sum(posf)

    li = lax.broadcasted_iota(jnp.int32, (1, 128), 1)
    v = (jnp.where(li == 0, part_l, 0.0) + jnp.where(li == 1, part_r, 0.0)
         + jnp.where(li == 2, part_ce, 0.0) + jnp.where(li == 3, npos, 0.0))
    ri = lax.broadcasted_iota(jnp.int32, (acc_ref.shape[0], 128), 0)
    acc_ref[...] += jnp.where(ri == b, 1.0, 0.0) * v


def _select_kernel(ce_ref, acc_ref, out_ref):
    cn = ce_ref[...]
    accv = acc_ref[...]
    P = cn.shape[1]
    npos = accv[:, 3:4]
    kf = jnp.minimum(_NEGPOS * npos, float(P - 1))
    bits = lax.bitcast_convert_type(cn, jnp.int32)

    def body(_, carry):
        lo, hi = carry
        mid = lo + (hi - lo + 1) // 2
        cnt = jnp.sum((bits >= mid).astype(jnp.float32), axis=1, keepdims=True)
        pred = cnt >= kf
        return jnp.where(pred, mid, lo), jnp.where(pred, hi, mid - 1)

    lo = jnp.zeros((cn.shape[0], 1), jnp.int32)
    hi = jnp.full((cn.shape[0], 1), 0x7F800000, jnp.int32)
    lo, hi = lax.fori_loop(0, 31, body, (lo, hi))
    thr_f = lax.bitcast_convert_type(lo, jnp.float32)
    gt = bits > lo
    cnt_gt = jnp.sum(gt.astype(jnp.float32), axis=1, keepdims=True)
    topk = (jnp.sum(jnp.where(gt, cn, 0.0), axis=1, keepdims=True)
            + (kf - cnt_gt) * thr_f)
    n_total = jnp.sum(npos)
    loss_l = jnp.sum(accv[:, 0:1]) / n_total
    loss_r = jnp.sum(accv[:, 1:2]) / n_total
    loss_c = jnp.sum(accv[:, 2:3] + topk) / n_total
    ri = lax.broadcasted_iota(jnp.int32, out_ref.shape, 0)
    ci = lax.broadcasted_iota(jnp.int32, out_ref.shape, 1)
    r0 = ri == 0
    out_ref[...] = (jnp.where(r0 & (ci == 0), loss_l, 0.0)
                    + jnp.where(r0 & (ci == 1), loss_c, 0.0)
                    + jnp.where(r0 & (ci == 2), loss_r, 0.0))


def kernel(loc_data, conf_data, regr_data, priors, t_coords, t_labels, t_regr):
    B, P, C = conf_data.shape
    O = t_coords.shape[1]
    priors_t = priors.T
    lbl3 = t_labels.reshape(B, O, 1)

    conf_t, g_row, rt_row = pl.pallas_call(
        _match_kernel,
        grid=(B,),
        in_specs=[
            pl.BlockSpec((1, O, 4), lambda b: (b, 0, 0)),
            pl.BlockSpec((1, O, 1), lambda b: (b, 0, 0)),
            pl.BlockSpec((1, O, 1), lambda b: (b, 0, 0)),
            pl.BlockSpec((4, P), lambda b: (0, 0)),
        ],
        out_specs=[
            pl.BlockSpec((1, 1, P), lambda b: (b, 0, 0)),
            pl.BlockSpec((1, 2, P), lambda b: (b, 0, 0)),
            pl.BlockSpec((1, 1, P), lambda b: (b, 0, 0)),
        ],
        out_shape=[
            jax.ShapeDtypeStruct((B, 1, P), jnp.int32),
            jax.ShapeDtypeStruct((B, 2, P), jnp.float32),
            jax.ShapeDtypeStruct((B, 1, P), jnp.float32),
        ],
        scratch_shapes=[
            pltpu.VMEM((8, P), jnp.float32),
            pltpu.VMEM((8, P), jnp.int32),
        ],
    )(t_coords, lbl3, t_regr, priors_t)

    ct_col = conf_t.reshape(B, P, 1)
    g_col = g_row.transpose(0, 2, 1)
    rt_col = rt_row.reshape(B, P, 1)

    nt = P // _TILE
    ce_neg, acc = pl.pallas_call(
        _stream_kernel,
        grid=(B, nt),
        in_specs=[
            pl.BlockSpec((1, _TILE, C), lambda b, t: (b, t, 0)),
            pl.BlockSpec((1, _TILE, 2), lambda b, t: (b, t, 0)),
            pl.BlockSpec((1, _TILE, 1), lambda b, t: (b, t, 0)),
            pl.BlockSpec((1, _TILE, 1), lambda b, t: (b, t, 0)),
            pl.BlockSpec((1, _TILE, 2), lambda b, t: (b, t, 0)),
            pl.BlockSpec((1, _TILE, 1), lambda b, t: (b, t, 0)),
        ],
        out_specs=[
            pl.BlockSpec((1, _TILE, 1), lambda b, t: (b, t, 0)),
            pl.BlockSpec((B, 128), lambda b, t: (0, 0)),
        ],
        out_shape=[
            jax.ShapeDtypeStruct((B, P, 1), jnp.float32),
            jax.ShapeDtypeStruct((B, 128), jnp.float32),
        ],
    )(conf_data, loc_data, regr_data, ct_col, g_col, rt_col)

    out = pl.pallas_call(
        _select_kernel,
        in_specs=[
            pl.BlockSpec((B, P), lambda: (0, 0)),
            pl.BlockSpec((B, 128), lambda: (0, 0)),
        ],
        out_specs=pl.BlockSpec((8, 128), lambda: (0, 0)),
        out_shape=jax.ShapeDtypeStruct((8, 128), jnp.float32),
    )(ce_neg.reshape(B, P), acc)

    return (out[0, 0], out[0, 1], out[0, 2])


# speedup vs baseline: 24.3800x; 2.3266x over previous
import jax
import jax.numpy as jnp
from jax import lax
from jax.experimental import pallas as pl
from jax.experimental.pallas import tpu as pltpu

_THRESH = 0.5
_NEGPOS = 3.0
_VAR = 0.1
_CHUNK = 4096
_TILE = 4096
_LANE = 128


def _match_kernel(tc_ref, tbl_ref, pt_ref, conf_ref, g_ref, rt_ref,
                  btm_scr, bti_scr):
    O = tc_ref.shape[1]
    P = pt_ref.shape[1]
    nch = P // _CHUNK
    tc = tc_ref[0]
    ax1, ay1 = tc[:, 0:1], tc[:, 1:2]
    ax2, ay2 = tc[:, 2:3], tc[:, 3:4]
    area_a = (ax2 - ax1) * (ay2 - ay1)
    row_i = lax.broadcasted_iota(jnp.int32, (O, _CHUNK), 0)
    lane_i = lax.broadcasted_iota(jnp.int32, (O, _CHUNK), 1)

    bp_max = jnp.full((O, 1), -1.0, jnp.float32)
    bp_idx = jnp.zeros((O, 1), jnp.int32)
    for c in range(nch):
        sl = slice(c * _CHUNK, (c + 1) * _CHUNK)
        pcx, pcy = pt_ref[0:1, sl], pt_ref[1:2, sl]
        pw, ph = pt_ref[2:3, sl], pt_ref[3:4, sl]
        bx1, by1 = pcx - pw / 2, pcy - ph / 2
        bx2, by2 = pcx + pw / 2, pcy + ph / 2
        wid = jnp.maximum(jnp.minimum(ax2, bx2) - jnp.maximum(ax1, bx1), 0.0)
        hei = jnp.maximum(jnp.minimum(ay2, by2) - jnp.maximum(ay1, by1), 0.0)
        inter = wid * hei
        area_b = (bx2 - bx1) * (by2 - by1)
        ovl = inter / (area_a + area_b - inter)
        cmax = jnp.max(ovl, axis=1, keepdims=True)
        cidx = jnp.min(jnp.where(ovl == cmax, lane_i, P), axis=1,
                       keepdims=True) + c * _CHUNK
        upd = cmax > bp_max
        bp_idx = jnp.where(upd, cidx, bp_idx)
        bp_max = jnp.where(upd, cmax, bp_max)
        btm = jnp.max(ovl, axis=0, keepdims=True)
        bti = jnp.min(jnp.where(ovl == btm, row_i, O), axis=0, keepdims=True)
        btm_scr[0:1, sl] = btm
        bti_scr[0:1, sl] = bti

    tbl = tbl_ref[0]
    for c in range(nch):
        sl = slice(c * _CHUNK, (c + 1) * _CHUNK)
        btm = btm_scr[0:1, sl]
        bti = bti_scr[0:1, sl]
        lane_g = lane_i + c * _CHUNK
        F = lane_g == bp_idx
        ch_o = jnp.max(jnp.where(F, row_i, -1), axis=0, keepdims=True)
        forced = ch_o >= 0
        fidx = jnp.where(forced, ch_o, bti)
        fovl = jnp.where(forced, 2.0, btm)
        eq2f = (row_i == fidx).astype(jnp.float32)
        m = jax.lax.dot(tbl, eq2f,
                        precision=jax.lax.Precision.HIGHEST,
                        preferred_element_type=jnp.float32)
        conf = jnp.where(fovl < _THRESH, 0.0, m[0:1])
        pcx, pcy = pt_ref[0:1, sl], pt_ref[1:2, sl]
        pw, ph = pt_ref[2:3, sl], pt_ref[3:4, sl]
        conf_ref[0, 0:1, sl] = conf.astype(jnp.int32)
        g_ref[0, 0:1, sl] = (m[1:2] - pcx) / (_VAR * pw)
        g_ref[0, 1:2, sl] = (m[2:3] - pcy) / (_VAR * ph)
        rt_ref[0, 0:1, sl] = m[3:4]


def _stream_kernel(conf_ref, lx_ref, ly_ref, rd_ref, ct_ref, gx_ref, gy_ref,
                   rt_ref, ce_ref, acc_ref):
    b = pl.program_id(0)
    t = pl.program_id(1)
    G, L, C = conf_ref.shape[1], conf_ref.shape[2], conf_ref.shape[3]

    @pl.when(jnp.logical_and(b == 0, t == 0))
    def _init():
        acc_ref[...] = jnp.zeros_like(acc_ref)

    x = conf_ref[0]
    m = jnp.max(x, axis=2)
    s = jnp.sum(jnp.exp(x - m[:, :, None]), axis=2)
    lse = m + jnp.log(s)
    ct = ct_ref[0]
    cls_i = lax.broadcasted_iota(jnp.int32, (G, L, C), 2)
    gathered = jnp.sum(jnp.where(cls_i == ct[:, :, None], x, 0.0), axis=2)
    ce = lse - gathered
    pos = ct > 0
    posf = pos.astype(jnp.float32)
    ce_ref[0] = jnp.where(pos, 0.0, ce)

    dx = lx_ref[0] - gx_ref[0]
    dy = ly_ref[0] - gy_ref[0]
    adx, ady = jnp.abs(dx), jnp.abs(dy)
    sl1 = (jnp.where(adx < 1.0, 0.5 * dx * dx, adx - 0.5)
           + jnp.where(ady < 1.0, 0.5 * dy * dy, ady - 0.5))
    part_l = jnp.sum(sl1 * posf)
    part_r = jnp.sum(jnp.abs(rd_ref[0] - rt_ref[0]) * posf)
    part_ce = jnp.sum(ce * posf)
    npos = jnp.sum(posf)

    li = lax.broadcasted_iota(jnp.int32, (1, 128), 1)
    v = (jnp.where(li == 0, part_l, 0.0) + jnp.where(li == 1, part_r, 0.0)
         + jnp.where(li == 2, part_ce, 0.0) + jnp.where(li == 3, npos, 0.0))
    ri = lax.broadcasted_iota(jnp.int32, (acc_ref.shape[0], 128), 0)
    acc_ref[...] += jnp.where(ri == b, 1.0, 0.0) * v


def _select_kernel(ce_ref, acc_ref, out_ref):
    cn = ce_ref[...]
    accv = acc_ref[...]
    P = cn.shape[1]
    npos = accv[:, 3:4]
    kf = jnp.minimum(_NEGPOS * npos, float(P - 1))
    bits = lax.bitcast_convert_type(cn, jnp.int32)

    def body(_, carry):
        lo, hi = carry
        mid = lo + (hi - lo + 1) // 2
        cnt = jnp.sum((bits >= mid).astype(jnp.float32), axis=1, keepdims=True)
        pred = cnt >= kf
        return jnp.where(pred, mid, lo), jnp.where(pred, hi, mid - 1)

    lo = jnp.zeros((cn.shape[0], 1), jnp.int32)
    hi = jnp.full((cn.shape[0], 1), 0x7F800000, jnp.int32)
    lo, hi = lax.fori_loop(0, 31, body, (lo, hi))
    thr_f = lax.bitcast_convert_type(lo, jnp.float32)
    gt = bits > lo
    cnt_gt = jnp.sum(gt.astype(jnp.float32), axis=1, keepdims=True)
    topk = (jnp.sum(jnp.where(gt, cn, 0.0), axis=1, keepdims=True)
            + (kf - cnt_gt) * thr_f)
    n_total = jnp.sum(npos)
    loss_l = jnp.sum(accv[:, 0:1]) / n_total
    loss_r = jnp.sum(accv[:, 1:2]) / n_total
    loss_c = jnp.sum(accv[:, 2:3] + topk) / n_total
    ri = lax.broadcasted_iota(jnp.int32, out_ref.shape, 0)
    ci = lax.broadcasted_iota(jnp.int32, out_ref.shape, 1)
    r0 = ri == 0
    out_ref[...] = (jnp.where(r0 & (ci == 0), loss_l, 0.0)
                    + jnp.where(r0 & (ci == 1), loss_c, 0.0)
                    + jnp.where(r0 & (ci == 2), loss_r, 0.0))


def kernel(loc_data, conf_data, regr_data, priors, t_coords, t_labels, t_regr):
    B, P, C = conf_data.shape
    O = t_coords.shape[1]
    priors_t = priors.T
    tcx = (t_coords[:, :, 0] + t_coords[:, :, 2]) * 0.5
    tcy = (t_coords[:, :, 1] + t_coords[:, :, 3]) * 0.5
    tbl = jnp.stack([t_labels.astype(jnp.float32) + 1.0, tcx, tcy,
                     t_regr[:, :, 0]], axis=1)

    conf_t, g_row, rt_row = pl.pallas_call(
        _match_kernel,
        grid=(B,),
        in_specs=[
            pl.BlockSpec((1, O, 4), lambda b: (b, 0, 0)),
            pl.BlockSpec((1, 4, O), lambda b: (b, 0, 0)),
            pl.BlockSpec((4, P), lambda b: (0, 0)),
        ],
        out_specs=[
            pl.BlockSpec((1, 1, P), lambda b: (b, 0, 0)),
            pl.BlockSpec((1, 2, P), lambda b: (b, 0, 0)),
            pl.BlockSpec((1, 1, P), lambda b: (b, 0, 0)),
        ],
        out_shape=[
            jax.ShapeDtypeStruct((B, 1, P), jnp.int32),
            jax.ShapeDtypeStruct((B, 2, P), jnp.float32),
            jax.ShapeDtypeStruct((B, 1, P), jnp.float32),
        ],
        scratch_shapes=[
            pltpu.VMEM((8, P), jnp.float32),
            pltpu.VMEM((8, P), jnp.int32),
        ],
    )(t_coords, tbl, priors_t)

    PG = P // _LANE
    TG = _TILE // _LANE
    conf4 = conf_data.reshape(B, PG, _LANE, C)
    ct_g = conf_t.reshape(B, PG, _LANE)
    gx_g = g_row[:, 0, :].reshape(B, PG, _LANE)
    gy_g = g_row[:, 1, :].reshape(B, PG, _LANE)
    rt_g = rt_row.reshape(B, PG, _LANE)
    lx_g = loc_data[:, :, 0].reshape(B, PG, _LANE)
    ly_g = loc_data[:, :, 1].reshape(B, PG, _LANE)
    rd_g = regr_data.reshape(B, PG, _LANE)

    nt = P // _TILE
    spec3 = pl.BlockSpec((1, TG, _LANE), lambda b, t: (b, t, 0))
    ce_neg, acc = pl.pallas_call(
        _stream_kernel,
        grid=(B, nt),
        in_specs=[
            pl.BlockSpec((1, TG, _LANE, C), lambda b, t: (b, t, 0, 0)),
            spec3, spec3, spec3, spec3, spec3, spec3, spec3,
        ],
        out_specs=[
            pl.BlockSpec((1, TG, _LANE), lambda b, t: (b, t, 0)),
            pl.BlockSpec((B, 128), lambda b, t: (0, 0)),
        ],
        out_shape=[
            jax.ShapeDtypeStruct((B, PG, _LANE), jnp.float32),
            jax.ShapeDtypeStruct((B, 128), jnp.float32),
        ],
    )(conf4, lx_g, ly_g, rd_g, ct_g, gx_g, gy_g, rt_g)

    out = pl.pallas_call(
        _select_kernel,
        in_specs=[
            pl.BlockSpec((B, P), lambda: (0, 0)),
            pl.BlockSpec((B, 128), lambda: (0, 0)),
        ],
        out_specs=pl.BlockSpec((8, 128), lambda: (0, 0)),
        out_shape=jax.ShapeDtypeStruct((8, 128), jnp.float32),
    )(ce_neg.reshape(B, P), acc)

    return (out[0, 0], out[0, 1], out[0, 2])
